# baseline (speedup 1.0000x reference)
import functools
import jax
import jax.numpy as jnp
from jax import lax
from jax.experimental import pallas as pl
from jax.experimental.pallas import tpu as pltpu
from jax.experimental.pallas import tpu_sc as plsc

PAD_TOKEN = 0


def _rank_body(inp_ref, noise_ref, rank_ref):
    inp = inp_ref[...]
    noise = noise_ref[...]
    B, S = inp.shape
    pos = jnp.where(inp == PAD_TOKEN, jnp.inf, noise)
    a = pos[:, :, None]
    b = pos[:, None, :]
    k_i = lax.broadcasted_iota(jnp.int32, (1, S, S), 2)
    j_i = lax.broadcasted_iota(jnp.int32, (1, S, S), 1)
    tie = k_i < j_i
    cmp = (b < a) | ((b == a) & tie)
    rank_f = jnp.sum(cmp.astype(jnp.float32), axis=2)
    rank_ref[...] = (rank_f + 0.5).astype(jnp.int32)


def _compute_ranks(input, noise):
    BATCH, SEQ = input.shape
    B_BLK = 8
    return pl.pallas_call(
        _rank_body,
        grid=(BATCH // B_BLK,),
        in_specs=[
            pl.BlockSpec((B_BLK, SEQ), lambda i: (i, 0)),
            pl.BlockSpec((B_BLK, SEQ), lambda i: (i, 0)),
        ],
        out_specs=pl.BlockSpec((B_BLK, SEQ), lambda i: (i, 0)),
        out_shape=jax.ShapeDtypeStruct((BATCH, SEQ), jnp.int32),
    )(input, noise)


def _pair_table_body(pe_ref, out_ref):
    i = pl.program_id(0)
    S, E = pe_ref.shape
    R = out_ref.shape[0] // S
    pe_r = pe_ref[pl.ds(R * i, R), :]
    left = jnp.broadcast_to(pe_r[:, None, :], (R, S, E))
    right = jnp.broadcast_to(pe_ref[...][None, :, :], (R, S, E))
    out_ref[:, :E] = left.reshape(R * S, E)
    out_ref[:, E:] = right.reshape(R * S, E)


def _build_pair_table(pe_s, r_blk=8):
    S, E = pe_s.shape
    return pl.pallas_call(
        _pair_table_body,
        grid=(S // r_blk,),
        in_specs=[pl.BlockSpec((S, E), lambda i: (0, 0))],
        out_specs=pl.BlockSpec((r_blk * S, 2 * E), lambda i: (i, 0)),
        out_shape=jax.ShapeDtypeStruct((S * S, 2 * E), jnp.float32),
    )(pe_s)


def _make_sc_gather(batch, seq, emb, nc, ns, b_chunk):
    nw = nc * ns
    s2 = seq // 2
    assert batch % (nw * 2 * b_chunk) == 0
    b_per_w = batch // nw
    n_chunks = b_per_w // b_chunk
    rows_per_chunk = b_chunk * s2
    mesh = plsc.VectorSubcoreMesh(core_axis_name="c", subcore_axis_name="s")

    @functools.partial(
        pl.kernel,
        mesh=mesh,
        out_type=jax.ShapeDtypeStruct((batch, s2, 2 * emb), jnp.float32),
        scratch_types=[
            pltpu.VMEM((b_per_w * s2,), jnp.int32),
            pltpu.VMEM((rows_per_chunk, 2 * emb), jnp.float32),
            pltpu.VMEM((rows_per_chunk, 2 * emb), jnp.float32),
            pltpu.SemaphoreType.DMA,
            pltpu.SemaphoreType.DMA,
        ],
    )
    def sc_gather(idx_hbm, table_hbm, out_hbm, idx_v, buf0, buf1, g0, g1):
        wid = lax.axis_index("s") * nc + lax.axis_index("c")
        base_b = wid * b_per_w
        pltpu.sync_copy(idx_hbm.at[pl.ds(base_b * s2, b_per_w * s2)], idx_v)

        def gather(c, buf, sem):
            return pltpu.async_copy(
                table_hbm.at[idx_v.at[pl.ds(c * rows_per_chunk, rows_per_chunk)]],
                buf,
                sem,
            )

        def wait_gather(buf, sem):
            pltpu.make_async_copy(
                table_hbm.at[pl.ds(0, rows_per_chunk)], buf, sem
            ).wait()

        def write_chunk(c, buf):
            for i in range(b_chunk):
                pltpu.sync_copy(
                    buf.at[pl.ds(i * s2, s2)],
                    out_hbm.at[base_b + c * b_chunk + i],
                )

        gather(0, buf0, g0)

        def body(t, carry):
            c0 = 2 * t
            c1 = 2 * t + 1
            gather(c1, buf1, g1)
            wait_gather(buf0, g0)
            write_chunk(c0, buf0)

            @pl.when(t < n_chunks // 2 - 1)
            def _():
                gather(c0 + 2, buf0, g0)

            wait_gather(buf1, g1)
            write_chunk(c1, buf1)
            return carry

        lax.fori_loop(0, n_chunks // 2, body, 0)

    return sc_gather


def _repack_body(in_ref, out_ref):
    B, S2, E2 = in_ref.shape
    E = E2 // 2
    x = in_ref[...]
    y = jnp.concatenate([x[:, :, :E], x[:, :, E:]], axis=1)
    out_ref[...] = y.reshape(B * 2 * S2, E)


def _repack(paired, batch, seq, emb):
    B_BLK = 8
    s2 = seq // 2
    return pl.pallas_call(
        _repack_body,
        grid=(batch // B_BLK,),
        in_specs=[pl.BlockSpec((B_BLK, s2, 2 * emb), lambda i: (i, 0, 0))],
        out_specs=pl.BlockSpec((B_BLK * seq, emb), lambda i: (i, 0)),
        out_shape=jax.ShapeDtypeStruct((batch * seq, emb), jnp.float32),
    )(paired)


def kernel(input, pe, noise):
    BATCH, SEQ = input.shape
    EMB = pe.shape[1]
    ranks = _compute_ranks(input, noise)
    pair_idx = (
        ranks[:, : SEQ // 2] * SEQ + ranks[:, SEQ // 2 :]
    ).reshape(BATCH * SEQ // 2)
    table = _build_pair_table(pe[:SEQ])
    info = plsc.get_sparse_core_info()
    gather = _make_sc_gather(
        BATCH, SEQ, EMB, info.num_cores, info.num_subcores, 4
    )
    paired = gather(pair_idx, table)
    return jnp.concatenate([paired[:, :, :EMB], paired[:, :, EMB:]], axis=1)


# speedup vs baseline: 1.2740x; 1.2740x over previous
import functools
import jax
import jax.numpy as jnp
from jax import lax
from jax.experimental import pallas as pl
from jax.experimental.pallas import tpu as pltpu
from jax.experimental.pallas import tpu_sc as plsc

PAD_TOKEN = 0


def _rank_body(inp_ref, noise_ref, pair_ref):
    inp = inp_ref[...]
    noise = noise_ref[...]
    B, S = inp.shape
    pos = jnp.where(inp == PAD_TOKEN, jnp.inf, noise)
    a = pos[:, :, None]
    b = pos[:, None, :]
    k_i = lax.broadcasted_iota(jnp.int32, (1, S, S), 2)
    j_i = lax.broadcasted_iota(jnp.int32, (1, S, S), 1)
    tie = k_i < j_i
    cmp = (b < a) | ((b == a) & tie)
    rank_f = jnp.sum(cmp.astype(jnp.float32), axis=2)
    kk = lax.broadcasted_iota(jnp.int32, (S, S // 2), 0)
    pp = lax.broadcasted_iota(jnp.int32, (S, S // 2), 1)
    sel = jnp.where(kk == 2 * pp, float(S), 0.0) + jnp.where(
        kk == 2 * pp + 1, 1.0, 0.0
    )
    pair = jnp.dot(rank_f, sel, preferred_element_type=jnp.float32)
    pair_ref[...] = (pair + 0.5).astype(jnp.int32)


def _compute_pair_idx(input, noise):
    BATCH, SEQ = input.shape
    B_BLK = 16
    return pl.pallas_call(
        _rank_body,
        grid=(BATCH // B_BLK,),
        in_specs=[
            pl.BlockSpec((B_BLK, SEQ), lambda i: (i, 0)),
            pl.BlockSpec((B_BLK, SEQ), lambda i: (i, 0)),
        ],
        out_specs=pl.BlockSpec((B_BLK, SEQ // 2), lambda i: (i, 0)),
        out_shape=jax.ShapeDtypeStruct((BATCH, SEQ // 2), jnp.int32),
    )(input, noise)


def _pair_table_body(pe_ref, out_ref):
    i = pl.program_id(0)
    S, E = pe_ref.shape
    R = out_ref.shape[0] // S
    pe_r = pe_ref[pl.ds(R * i, R), :]
    left = jnp.broadcast_to(pe_r[:, None, :], (R, S, E))
    right = jnp.broadcast_to(pe_ref[...][None, :, :], (R, S, E))
    out_ref[:, :E] = left.reshape(R * S, E)
    out_ref[:, E:] = right.reshape(R * S, E)


def _build_pair_table(pe_s, r_blk=8):
    S, E = pe_s.shape
    return pl.pallas_call(
        _pair_table_body,
        grid=(S // r_blk,),
        in_specs=[pl.BlockSpec((S, E), lambda i: (0, 0))],
        out_specs=pl.BlockSpec((r_blk * S, 2 * E), lambda i: (i, 0)),
        out_shape=jax.ShapeDtypeStruct((S * S, 2 * E), jnp.float32),
    )(pe_s)


def _make_sc_gather(n_idx, emb2, nc, ns, chunk):
    nw = nc * ns
    assert n_idx % (nw * chunk) == 0
    b_per_w = n_idx // nw
    n_chunks = b_per_w // chunk
    mesh = plsc.VectorSubcoreMesh(core_axis_name="c", subcore_axis_name="s")

    @functools.partial(
        pl.kernel,
        mesh=mesh,
        out_type=jax.ShapeDtypeStruct((n_idx, emb2), jnp.float32),
        scratch_types=[
            pltpu.VMEM((b_per_w,), jnp.int32),
            pltpu.VMEM((chunk, emb2), jnp.float32),
            pltpu.VMEM((chunk, emb2), jnp.float32),
            pltpu.SemaphoreType.DMA,
            pltpu.SemaphoreType.DMA,
            pltpu.SemaphoreType.DMA,
            pltpu.SemaphoreType.DMA,
        ],
    )
    def sc_gather(idx_hbm, table_hbm, out_hbm, idx_v, rows0, rows1, g0, g1, w0, w1):
        wid = lax.axis_index("s") * nc + lax.axis_index("c")
        base = wid * b_per_w
        pltpu.sync_copy(idx_hbm.at[pl.ds(base, b_per_w)], idx_v)
        rows = [rows0, rows1]
        gsem = [g0, g1]
        wsem = [w0, w1]
        pend_g = [None, None]
        pend_w = [None, None]
        pend_g[0] = pltpu.async_copy(
            table_hbm.at[idx_v.at[pl.ds(0, chunk)]], rows[0], gsem[0]
        )
        for c in range(n_chunks):
            i = c % 2
            ni = (c + 1) % 2
            if c + 1 < n_chunks:
                if pend_w[ni] is not None:
                    pend_w[ni].wait()
                    pend_w[ni] = None
                pend_g[ni] = pltpu.async_copy(
                    table_hbm.at[idx_v.at[pl.ds((c + 1) * chunk, chunk)]],
                    rows[ni],
                    gsem[ni],
                )
            pend_g[i].wait()
            pend_g[i] = None
            pend_w[i] = pltpu.async_copy(
                rows[i], out_hbm.at[pl.ds(base + c * chunk, chunk)], wsem[i]
            )
        for i in range(2):
            if pend_w[i] is not None:
                pend_w[i].wait()

    return sc_gather


def kernel(input, pe, noise):
    BATCH, SEQ = input.shape
    EMB = pe.shape[1]
    pair_idx = _compute_pair_idx(input, noise).reshape(BATCH * SEQ // 2)
    table = _build_pair_table(pe[:SEQ])
    info = plsc.get_sparse_core_info()
    gather = _make_sc_gather(
        BATCH * SEQ // 2, 2 * EMB, info.num_cores, info.num_subcores, 320
    )
    out = gather(pair_idx, table)
    return out.reshape(BATCH, SEQ, EMB)


# speedup vs baseline: 1.3074x; 1.0263x over previous
import functools
import jax
import jax.numpy as jnp
from jax import lax
from jax.experimental import pallas as pl
from jax.experimental.pallas import tpu as pltpu
from jax.experimental.pallas import tpu_sc as plsc

PAD_TOKEN = 0


def _rank_body(inp_ref, noise_ref, pair_ref):
    inp = inp_ref[...]
    noise = noise_ref[...]
    B, S = inp.shape
    pos = jnp.where(inp == PAD_TOKEN, jnp.inf, noise)
    a = pos[:, :, None]
    b = pos[:, None, :]
    k_i = lax.broadcasted_iota(jnp.int32, (1, S, S), 2)
    j_i = lax.broadcasted_iota(jnp.int32, (1, S, S), 1)
    tie = k_i < j_i
    cmp = (b < a) | ((b == a) & tie)
    rank_f = jnp.sum(cmp.astype(jnp.float32), axis=2)
    kk = lax.broadcasted_iota(jnp.int32, (S, S // 2), 0)
    pp = lax.broadcasted_iota(jnp.int32, (S, S // 2), 1)
    sel = jnp.where(kk == 2 * pp, float(S), 0.0) + jnp.where(
        kk == 2 * pp + 1, 1.0, 0.0
    )
    pair = jnp.dot(rank_f, sel, preferred_element_type=jnp.float32)
    pair_ref[...] = (pair + 0.5).astype(jnp.int32)


def _compute_pair_idx(input, noise):
    BATCH, SEQ = input.shape
    B_BLK = 32
    return pl.pallas_call(
        _rank_body,
        grid=(BATCH // B_BLK,),
        in_specs=[
            pl.BlockSpec((B_BLK, SEQ), lambda i: (i, 0)),
            pl.BlockSpec((B_BLK, SEQ), lambda i: (i, 0)),
        ],
        out_specs=pl.BlockSpec((B_BLK, SEQ // 2), lambda i: (i, 0)),
        out_shape=jax.ShapeDtypeStruct((BATCH, SEQ // 2), jnp.int32),
    )(input, noise)


def _pair_table_body(pe_ref, out_ref):
    i = pl.program_id(0)
    S, E = pe_ref.shape
    R = out_ref.shape[0] // S
    pe_r = pe_ref[pl.ds(R * i, R), :]
    left = jnp.broadcast_to(pe_r[:, None, :], (R, S, E))
    right = jnp.broadcast_to(pe_ref[...][None, :, :], (R, S, E))
    out_ref[:, :E] = left.reshape(R * S, E)
    out_ref[:, E:] = right.reshape(R * S, E)


def _build_pair_table(pe_s, r_blk=8):
    S, E = pe_s.shape
    return pl.pallas_call(
        _pair_table_body,
        grid=(S // r_blk,),
        in_specs=[pl.BlockSpec((S, E), lambda i: (0, 0))],
        out_specs=pl.BlockSpec((r_blk * S, 2 * E), lambda i: (i, 0)),
        out_shape=jax.ShapeDtypeStruct((S * S, 2 * E), jnp.float32),
    )(pe_s)


def _make_sc_gather(n_idx, emb2, nc, ns, chunk):
    nw = nc * ns
    assert n_idx % (nw * chunk) == 0
    b_per_w = n_idx // nw
    n_chunks = b_per_w // chunk
    mesh = plsc.VectorSubcoreMesh(core_axis_name="c", subcore_axis_name="s")

    @functools.partial(
        pl.kernel,
        mesh=mesh,
        out_type=jax.ShapeDtypeStruct((n_idx, emb2), jnp.float32),
        scratch_types=[
            pltpu.VMEM((b_per_w,), jnp.int32),
            pltpu.VMEM((chunk, emb2), jnp.float32),
            pltpu.VMEM((chunk, emb2), jnp.float32),
            pltpu.SemaphoreType.DMA,
            pltpu.SemaphoreType.DMA,
            pltpu.SemaphoreType.DMA,
            pltpu.SemaphoreType.DMA,
        ],
    )
    def sc_gather(idx_hbm, table_hbm, out_hbm, idx_v, rows0, rows1, g0, g1, w0, w1):
        wid = lax.axis_index("s") * nc + lax.axis_index("c")
        base = wid * b_per_w
        pltpu.sync_copy(idx_hbm.at[pl.ds(base, b_per_w)], idx_v)
        rows = [rows0, rows1]
        gsem = [g0, g1]
        wsem = [w0, w1]
        pend_g = [None, None]
        pend_w = [None, None]
        pend_g[0] = pltpu.async_copy(
            table_hbm.at[idx_v.at[pl.ds(0, chunk)]], rows[0], gsem[0]
        )
        for c in range(n_chunks):
            i = c % 2
            ni = (c + 1) % 2
            if c + 1 < n_chunks:
                if pend_w[ni] is not None:
                    pend_w[ni].wait()
                    pend_w[ni] = None
                pend_g[ni] = pltpu.async_copy(
                    table_hbm.at[idx_v.at[pl.ds((c + 1) * chunk, chunk)]],
                    rows[ni],
                    gsem[ni],
                )
            pend_g[i].wait()
            pend_g[i] = None
            pend_w[i] = pltpu.async_copy(
                rows[i], out_hbm.at[pl.ds(base + c * chunk, chunk)], wsem[i]
            )
        for i in range(2):
            if pend_w[i] is not None:
                pend_w[i].wait()

    return sc_gather


def kernel(input, pe, noise):
    BATCH, SEQ = input.shape
    EMB = pe.shape[1]
    pair_idx = _compute_pair_idx(input, noise).reshape(BATCH * SEQ // 2)
    table = _build_pair_table(pe[:SEQ])
    info = plsc.get_sparse_core_info()
    gather = _make_sc_gather(
        BATCH * SEQ // 2, 2 * EMB, info.num_cores, info.num_subcores, 320
    )
    out = gather(pair_idx, table)
    return out.reshape(BATCH, SEQ, EMB)


# speedup vs baseline: 1.3343x; 1.0206x over previous
import functools
import jax
import jax.numpy as jnp
from jax import lax
from jax.experimental import pallas as pl
from jax.experimental.pallas import tpu as pltpu
from jax.experimental.pallas import tpu_sc as plsc

PAD_TOKEN = 0


def _rank_body(inp_ref, noise_ref, pair_ref):
    inp = inp_ref[...]
    noise = noise_ref[...]
    B, S = inp.shape
    pos = jnp.where(inp == PAD_TOKEN, jnp.inf, noise)
    a = pos[:, :, None]
    b = pos[:, None, :]
    k_i = lax.broadcasted_iota(jnp.int32, (1, S, S), 2)
    j_i = lax.broadcasted_iota(jnp.int32, (1, S, S), 1)
    tie = k_i < j_i
    cmp = (b < a) | ((b == a) & tie)
    rank_f = jnp.sum(cmp.astype(jnp.float32), axis=2)
    kk = lax.broadcasted_iota(jnp.int32, (S, S // 2), 0)
    pp = lax.broadcasted_iota(jnp.int32, (S, S // 2), 1)
    sel = jnp.where(kk == 2 * pp, float(S), 0.0) + jnp.where(
        kk == 2 * pp + 1, 1.0, 0.0
    )
    pair = jnp.dot(rank_f, sel, preferred_element_type=jnp.float32)
    pair_ref[...] = (pair + 0.5).astype(jnp.int32)


def _compute_pair_idx(input, noise):
    BATCH, SEQ = input.shape
    B_BLK = 64
    return pl.pallas_call(
        _rank_body,
        grid=(BATCH // B_BLK,),
        in_specs=[
            pl.BlockSpec((B_BLK, SEQ), lambda i: (i, 0)),
            pl.BlockSpec((B_BLK, SEQ), lambda i: (i, 0)),
        ],
        out_specs=pl.BlockSpec((B_BLK, SEQ // 2), lambda i: (i, 0)),
        out_shape=jax.ShapeDtypeStruct((BATCH, SEQ // 2), jnp.int32),
    )(input, noise)


def _pair_table_body(pe_ref, out_ref):
    i = pl.program_id(0)
    S, E = pe_ref.shape
    R = out_ref.shape[0] // S
    pe_r = pe_ref[pl.ds(R * i, R), :]
    left = jnp.broadcast_to(pe_r[:, None, :], (R, S, E))
    right = jnp.broadcast_to(pe_ref[...][None, :, :], (R, S, E))
    out_ref[:, :E] = left.reshape(R * S, E)
    out_ref[:, E:] = right.reshape(R * S, E)


def _build_pair_table(pe_s, r_blk=8):
    S, E = pe_s.shape
    return pl.pallas_call(
        _pair_table_body,
        grid=(S // r_blk,),
        in_specs=[pl.BlockSpec((S, E), lambda i: (0, 0))],
        out_specs=pl.BlockSpec((r_blk * S, 2 * E), lambda i: (i, 0)),
        out_shape=jax.ShapeDtypeStruct((S * S, 2 * E), jnp.float32),
    )(pe_s)


def _make_sc_gather(n_idx, emb2, nc, ns, chunk):
    nw = nc * ns
    assert n_idx % (nw * chunk) == 0
    b_per_w = n_idx // nw
    n_chunks = b_per_w // chunk
    mesh = plsc.VectorSubcoreMesh(core_axis_name="c", subcore_axis_name="s")

    @functools.partial(
        pl.kernel,
        mesh=mesh,
        out_type=jax.ShapeDtypeStruct((n_idx, emb2), jnp.float32),
        scratch_types=[
            pltpu.VMEM((b_per_w,), jnp.int32),
            pltpu.VMEM((chunk, emb2), jnp.float32),
            pltpu.VMEM((chunk, emb2), jnp.float32),
            pltpu.SemaphoreType.DMA,
            pltpu.SemaphoreType.DMA,
            pltpu.SemaphoreType.DMA,
            pltpu.SemaphoreType.DMA,
        ],
    )
    def sc_gather(idx_hbm, table_hbm, out_hbm, idx_v, rows0, rows1, g0, g1, w0, w1):
        wid = lax.axis_index("s") * nc + lax.axis_index("c")
        base = wid * b_per_w
        pltpu.sync_copy(idx_hbm.at[pl.ds(base, b_per_w)], idx_v)
        rows = [rows0, rows1]
        gsem = [g0, g1]
        wsem = [w0, w1]
        pend_g = [None, None]
        pend_w = [None, None]
        pend_g[0] = pltpu.async_copy(
            table_hbm.at[idx_v.at[pl.ds(0, chunk)]], rows[0], gsem[0]
        )
        for c in range(n_chunks):
            i = c % 2
            ni = (c + 1) % 2
            if c + 1 < n_chunks:
                if pend_w[ni] is not None:
                    pend_w[ni].wait()
                    pend_w[ni] = None
                pend_g[ni] = pltpu.async_copy(
                    table_hbm.at[idx_v.at[pl.ds((c + 1) * chunk, chunk)]],
                    rows[ni],
                    gsem[ni],
                )
            pend_g[i].wait()
            pend_g[i] = None
            pend_w[i] = pltpu.async_copy(
                rows[i], out_hbm.at[pl.ds(base + c * chunk, chunk)], wsem[i]
            )
        for i in range(2):
            if pend_w[i] is not None:
                pend_w[i].wait()

    return sc_gather


def kernel(input, pe, noise):
    BATCH, SEQ = input.shape
    EMB = pe.shape[1]
    pair_idx = _compute_pair_idx(input, noise).reshape(BATCH * SEQ // 2)
    table = _build_pair_table(pe[:SEQ])
    info = plsc.get_sparse_core_info()
    gather = _make_sc_gather(
        BATCH * SEQ // 2, 2 * EMB, info.num_cores, info.num_subcores, 320
    )
    out = gather(pair_idx, table)
    return out.reshape(BATCH, SEQ, EMB)
